# baseline (speedup 1.0000x reference)
import functools

import jax
import jax.numpy as jnp
from jax import lax
from jax.experimental import pallas as pl
from jax.experimental.pallas import tpu as pltpu
from jax.experimental.pallas import tpu_sc as plsc

N_NODES = 10000
E_EDGES = 320000
D = 128
DH = D // 2
NC = 2
NS = 16
L = 16

CH = 64
NCHT = E_EDGES // CH
KK = (NCHT + NS - 1) // NS

NP = 10240
PERT = NP // NS


def _h_pre_body(h_ref, g_ref, b_ref, wa_ref, ba_ref, wb_ref, bb_ref,
                wc_ref, bc_ref, wd_ref, bd_ref,
                ah_ref, bh_ref, ch_ref, dh_ref):
    x = h_ref[...]
    mean = jnp.mean(x, axis=0, keepdims=True)
    xc = x - mean
    var = jnp.mean(xc * xc, axis=0, keepdims=True)
    xn = g_ref[...] * xc * jax.lax.rsqrt(var + 1e-5) + b_ref[...]
    ah_ref[...] = jnp.dot(xn, wa_ref[...], preferred_element_type=jnp.float32) + ba_ref[...]
    bh_ref[...] = jnp.dot(xn, wb_ref[...], preferred_element_type=jnp.float32) + bb_ref[...]
    ch_ref[...] = jnp.dot(xn, wc_ref[...], preferred_element_type=jnp.float32) + bc_ref[...]
    dh_ref[...] = jnp.dot(xn, wd_ref[...], preferred_element_type=jnp.float32) + bd_ref[...]


def _colstats_body(x_ref, o_ref):
    i = pl.program_id(0)
    x = x_ref[...]
    s1 = jnp.sum(x, axis=0, keepdims=True)
    s2 = jnp.sum(x * x, axis=0, keepdims=True)
    blk = jnp.concatenate([s1, s2, jnp.zeros((6, D), jnp.float32)], axis=0)

    @pl.when(i == 0)
    def _init():
        o_ref[...] = blk

    @pl.when(i != 0)
    def _acc():
        o_ref[...] += blk


def _e_pre_body(x_ref, st_ref, g_ref, b_ref, we_ref, be_ref, o_ref):
    x = x_ref[...]
    mean = st_ref[0:1, :] / E_EDGES
    var = st_ref[1:2, :] / E_EDGES - mean * mean
    xn = g_ref[...] * (x - mean) * jax.lax.rsqrt(var + 1e-5) + b_ref[...]
    ee = jnp.dot(xn, we_ref[...], preferred_element_type=jnp.float32) + be_ref[...]
    o_ref[0] = jnp.concatenate([ee[:, :DH], x[:, :DH]], axis=1)
    o_ref[1] = jnp.concatenate([ee[:, DH:], x[:, DH:]], axis=1)


def _h_post_body(ah_ref, acch_ref, accs_ref, hin_ref, g_ref, b_ref,
                 w1_ref, b1_ref, w2_ref, b2_ref, o_ref):
    hmid = ah_ref[...] + acch_ref[...] / (accs_ref[...] + 1e-10)
    h2 = hin_ref[...] + hmid
    mean = jnp.mean(h2, axis=0, keepdims=True)
    xc = h2 - mean
    var = jnp.mean(xc * xc, axis=0, keepdims=True)
    xn = g_ref[...] * xc * jax.lax.rsqrt(var + 1e-5) + b_ref[...]
    f = jnp.maximum(jnp.dot(xn, w1_ref[...], preferred_element_type=jnp.float32) + b1_ref[...], 0.0)
    o_ref[...] = h2 + jnp.dot(f, w2_ref[...], preferred_element_type=jnp.float32) + b2_ref[...]


def _e_post_body(xp_ref, s1_ref, s2_ref, g_ref, b_ref, w1_ref, b1_ref, w2_ref, b2_ref, o_ref):
    xp = xp_ref[...]
    nb = xp.shape[0]
    x_even = jnp.concatenate([xp[:, 0:DH], xp[:, D:D + DH]], axis=1)
    x_odd = jnp.concatenate([xp[:, DH:D], xp[:, D + DH:]], axis=1)
    x = jnp.stack([x_even, x_odd], axis=1).reshape(2 * nb, D)
    mean = jnp.sum(s1_ref[...], axis=0, keepdims=True) / E_EDGES
    var = jnp.sum(s2_ref[...], axis=0, keepdims=True) / E_EDGES - mean * mean
    xn = g_ref[...] * (x - mean) * jax.lax.rsqrt(var + 1e-5) + b_ref[...]
    f = jnp.maximum(jnp.dot(xn, w1_ref[...], preferred_element_type=jnp.float32) + b1_ref[...], 0.0)
    o_ref[...] = x + jnp.dot(f, w2_ref[...], preferred_element_type=jnp.float32) + b2_ref[...]


def _sc_edge_body(src_hbm, dst_hbm, cb_hbm, dh_hbm, ei_hbm,
                  e2_hbm, acc_hbm, st_hbm,
                  src_v, dst_v, cb_v, dh_v, ei_v, stage_v, e2p_v, st_v,
                  acc_sh,
                  sem0, sem1, sem2, sem3):
    c = lax.axis_index("c")
    s = lax.axis_index("s")
    c0 = pl.multiple_of(c * DH, DH)

    def _zst(i, _):
        for j in range(D // L):
            st_v[i, pl.ds(j * L, L)] = jnp.zeros((L,), jnp.float32)
        return 0
    lax.fori_loop(0, 8, _zst, 0)

    def _zrow(i, _):
        for j in range(D // L):
            stage_v[i, pl.ds(j * L, L)] = jnp.zeros((L,), jnp.float32)
        return 0
    lax.fori_loop(0, CH, _zrow, 0)

    def _zcopy(k, _):
        pltpu.sync_copy(stage_v, acc_sh.at[pl.ds(s * PERT + k * CH, CH)])
        return 0
    lax.fori_loop(0, PERT // CH, _zcopy, 0)
    plsc.subcore_barrier()

    off = c * N_NODES

    def _chunk(g):
        base = pl.multiple_of(g * CH, 8)
        pltpu.sync_copy(src_hbm.at[pl.ds(base, CH)], src_v)
        pltpu.sync_copy(dst_hbm.at[pl.ds(base, CH)], dst_v)

        def _adj(j, _):
            sl = pl.ds(j * L, L)
            src_v[sl] = src_v[sl] + off
            return 0
        lax.fori_loop(0, CH // L, _adj, 0)

        cp0 = pltpu.async_copy(cb_hbm.at[src_v], cb_v, sem0)
        cp1 = pltpu.async_copy(dh_hbm.at[dst_v], dh_v, sem1)
        cp2 = pltpu.async_copy(ei_hbm.at[c, pl.ds(base, CH)], ei_v, sem2)
        cp0.wait()
        cp1.wait()
        cp2.wait()

        def _pair(q, _):
            for r_par in range(2):
                r = q * 2 + r_par
                e2col = r_par * DH
                for j in range(DH // L):
                    jL = j * L
                    sl = pl.ds(jL, L)
                    t = cb_v[r, sl] + dh_v[r, pl.ds(c0 + jL, L)] + ei_v[r, sl]
                    sg = 1.0 / (1.0 + jnp.exp(-t))
                    e2 = t + ei_v[r, pl.ds(DH + jL, L)]
                    e2p_v[q, pl.ds(e2col + jL, L)] = e2
                    stage_v[r, sl] = cb_v[r, pl.ds(DH + jL, L)] * sg
                    stage_v[r, pl.ds(DH + jL, L)] = sg
                    plsc.addupdate(st_v.at[0, pl.ds(c0 + jL, L)], e2)
                    plsc.addupdate(st_v.at[1, pl.ds(c0 + jL, L)], e2 * e2)
            return 0
        lax.fori_loop(0, CH // 2, _pair, 0)

        cp3 = pltpu.async_copy(
            e2p_v, e2_hbm.at[pl.ds(pl.multiple_of(base // 2, 8), CH // 2),
                             pl.ds(c * D, D)], sem3)
        pltpu.sync_copy(stage_v, acc_sh.at[dst_v], add=True)
        cp3.wait()

    def _kk(kk, _):
        g = kk * NS + s

        @pl.when(g < NCHT)
        def _():
            _chunk(g)
        return 0

    lax.fori_loop(0, KK, _kk, 0)
    pltpu.sync_copy(st_v, st_hbm.at[c * NS + s])

    plsc.subcore_barrier()
    pltpu.sync_copy(acc_sh.at[pl.ds(s * PERT, PERT)],
                    acc_hbm.at[c, pl.ds(s * PERT, PERT)])


@functools.lru_cache(maxsize=None)
def _sc_kernels():
    mesh = plsc.VectorSubcoreMesh(core_axis_name="c", subcore_axis_name="s",
                                  num_cores=NC, num_subcores=NS)
    edge_pass = pl.kernel(
        _sc_edge_body,
        out_type=(
            jax.ShapeDtypeStruct((E_EDGES // 2, 2 * D), jnp.float32),
            jax.ShapeDtypeStruct((NC, NP, D), jnp.float32),
            jax.ShapeDtypeStruct((NC * NS, 8, D), jnp.float32),
        ),
        mesh=mesh,
        scratch_types=[
            pltpu.VMEM((CH,), jnp.int32),
            pltpu.VMEM((CH,), jnp.int32),
            pltpu.VMEM((CH, D), jnp.float32),
            pltpu.VMEM((CH, D), jnp.float32),
            pltpu.VMEM((CH, D), jnp.float32),
            pltpu.VMEM((CH, D), jnp.float32),
            pltpu.VMEM((CH // 2, D), jnp.float32),
            pltpu.VMEM((8, D), jnp.float32),
            pltpu.VMEM_SHARED((NP, D), jnp.float32),
            pltpu.SemaphoreType.DMA,
            pltpu.SemaphoreType.DMA,
            pltpu.SemaphoreType.DMA,
            pltpu.SemaphoreType.DMA,
        ],
    )
    return edge_pass


def kernel(h, e, edge_index, W_A, b_A, W_B, b_B, W_C, b_C, W_D, b_D, W_E, b_E,
           ffh_W1, ffh_b1, ffh_W2, ffh_b2, ffe_W1, ffe_b1, ffe_W2, ffe_b2,
           g1h_g, g1h_b, g1e_g, g1e_b, g2h_g, g2h_b, g2e_g, g2e_b):
    src = edge_index[0]
    dst = edge_index[1]
    row = lambda v: v.reshape(1, D)

    ah, bh, ch, dh = pl.pallas_call(
        _h_pre_body,
        out_shape=[jax.ShapeDtypeStruct((N_NODES, D), jnp.float32)] * 4,
    )(h, row(g1h_g), row(g1h_b), W_A, row(b_A), W_B, row(b_B),
      W_C, row(b_C), W_D, row(b_D))

    BLK = 2000
    grid = E_EDGES // BLK
    estats = pl.pallas_call(
        _colstats_body,
        grid=(grid,),
        in_specs=[pl.BlockSpec((BLK, D), lambda i: (i, 0))],
        out_specs=pl.BlockSpec((8, D), lambda i: (0, 0)),
        out_shape=jax.ShapeDtypeStruct((8, D), jnp.float32),
    )(e)

    ei = pl.pallas_call(
        _e_pre_body,
        grid=(grid,),
        in_specs=[
            pl.BlockSpec((BLK, D), lambda i: (i, 0)),
            pl.BlockSpec((8, D), lambda i: (0, 0)),
            pl.BlockSpec((1, D), lambda i: (0, 0)),
            pl.BlockSpec((1, D), lambda i: (0, 0)),
            pl.BlockSpec((D, D), lambda i: (0, 0)),
            pl.BlockSpec((1, D), lambda i: (0, 0)),
        ],
        out_specs=pl.BlockSpec((NC, BLK, D), lambda i: (0, i, 0)),
        out_shape=jax.ShapeDtypeStruct((NC, E_EDGES, D), jnp.float32),
    )(e, estats, row(g1e_g), row(g1e_b), W_E, row(b_E))

    cb = jnp.concatenate([jnp.concatenate([ch[:, :DH], bh[:, :DH]], axis=1),
                          jnp.concatenate([ch[:, DH:], bh[:, DH:]], axis=1)], axis=0)

    sc_edge = _sc_kernels()
    e2p, acc, st = sc_edge(src, dst, cb, dh, ei)

    acc_h = jnp.concatenate([acc[0, :N_NODES, :DH], acc[1, :N_NODES, :DH]], axis=1)
    acc_s = jnp.concatenate([acc[0, :N_NODES, DH:], acc[1, :N_NODES, DH:]], axis=1)

    h_out = pl.pallas_call(
        _h_post_body,
        out_shape=jax.ShapeDtypeStruct((N_NODES, D), jnp.float32),
    )(ah, acc_h, acc_s, h, row(g2h_g), row(g2h_b),
      ffh_W1, row(ffh_b1), ffh_W2, row(ffh_b2))

    e_out = pl.pallas_call(
        _e_post_body,
        grid=(grid,),
        in_specs=[
            pl.BlockSpec((BLK // 2, 2 * D), lambda i: (i, 0)),
            pl.BlockSpec((NC * NS, D), lambda i: (0, 0)),
            pl.BlockSpec((NC * NS, D), lambda i: (0, 0)),
            pl.BlockSpec((1, D), lambda i: (0, 0)),
            pl.BlockSpec((1, D), lambda i: (0, 0)),
            pl.BlockSpec((D, D), lambda i: (0, 0)),
            pl.BlockSpec((1, D), lambda i: (0, 0)),
            pl.BlockSpec((D, D), lambda i: (0, 0)),
            pl.BlockSpec((1, D), lambda i: (0, 0)),
        ],
        out_specs=pl.BlockSpec((BLK, D), lambda i: (i, 0)),
        out_shape=jax.ShapeDtypeStruct((E_EDGES, D), jnp.float32),
    )(e2p, st[:, 0, :], st[:, 1, :], row(g2e_g), row(g2e_b),
      ffe_W1, row(ffe_b1), ffe_W2, row(ffe_b2))

    return (h_out, e_out)


# speedup vs baseline: 1.2532x; 1.2532x over previous
import functools

import jax
import jax.numpy as jnp
from jax import lax
from jax.experimental import pallas as pl
from jax.experimental.pallas import tpu as pltpu
from jax.experimental.pallas import tpu_sc as plsc

N_NODES = 10000
E_EDGES = 320000
D = 128
DH = D // 2
NC = 2
NS = 16
L = 16

CH = 32
NCHT = E_EDGES // CH
KK = NCHT // NS

NP = 10240
PERT = NP // NS


def _h_pre_body(h_ref, g_ref, b_ref, wa_ref, ba_ref, wb_ref, bb_ref,
                wc_ref, bc_ref, wd_ref, bd_ref,
                ah_ref, bh_ref, ch_ref, dh_ref):
    x = h_ref[...]
    mean = jnp.mean(x, axis=0, keepdims=True)
    xc = x - mean
    var = jnp.mean(xc * xc, axis=0, keepdims=True)
    xn = g_ref[...] * xc * jax.lax.rsqrt(var + 1e-5) + b_ref[...]
    ah_ref[...] = jnp.dot(xn, wa_ref[...], preferred_element_type=jnp.float32) + ba_ref[...]
    bh_ref[...] = jnp.dot(xn, wb_ref[...], preferred_element_type=jnp.float32) + bb_ref[...]
    ch_ref[...] = jnp.dot(xn, wc_ref[...], preferred_element_type=jnp.float32) + bc_ref[...]
    dh_ref[...] = jnp.dot(xn, wd_ref[...], preferred_element_type=jnp.float32) + bd_ref[...]


def _colstats_body(x_ref, o_ref):
    i = pl.program_id(0)
    x = x_ref[...]
    s1 = jnp.sum(x, axis=0, keepdims=True)
    s2 = jnp.sum(x * x, axis=0, keepdims=True)
    blk = jnp.concatenate([s1, s2, jnp.zeros((6, D), jnp.float32)], axis=0)

    @pl.when(i == 0)
    def _init():
        o_ref[...] = blk

    @pl.when(i != 0)
    def _acc():
        o_ref[...] += blk


def _e_pre_body(x_ref, st_ref, g_ref, b_ref, we_ref, be_ref, o_ref):
    x = x_ref[...]
    mean = st_ref[0:1, :] / E_EDGES
    var = st_ref[1:2, :] / E_EDGES - mean * mean
    xn = g_ref[...] * (x - mean) * jax.lax.rsqrt(var + 1e-5) + b_ref[...]
    ee = jnp.dot(xn, we_ref[...], preferred_element_type=jnp.float32) + be_ref[...]
    o_ref[0] = jnp.concatenate([ee[:, :DH], x[:, :DH]], axis=1)
    o_ref[1] = jnp.concatenate([ee[:, DH:], x[:, DH:]], axis=1)


def _h_post_body(ah_ref, acch_ref, accs_ref, hin_ref, g_ref, b_ref,
                 w1_ref, b1_ref, w2_ref, b2_ref, o_ref):
    hmid = ah_ref[...] + acch_ref[...] / (accs_ref[...] + 1e-10)
    h2 = hin_ref[...] + hmid
    mean = jnp.mean(h2, axis=0, keepdims=True)
    xc = h2 - mean
    var = jnp.mean(xc * xc, axis=0, keepdims=True)
    xn = g_ref[...] * xc * jax.lax.rsqrt(var + 1e-5) + b_ref[...]
    f = jnp.maximum(jnp.dot(xn, w1_ref[...], preferred_element_type=jnp.float32) + b1_ref[...], 0.0)
    o_ref[...] = h2 + jnp.dot(f, w2_ref[...], preferred_element_type=jnp.float32) + b2_ref[...]


def _e_post_body(xp_ref, s1_ref, s2_ref, g_ref, b_ref, w1_ref, b1_ref, w2_ref, b2_ref, o_ref):
    xp = xp_ref[...]
    nb = xp.shape[0]
    x_even = jnp.concatenate([xp[:, 0:DH], xp[:, D:D + DH]], axis=1)
    x_odd = jnp.concatenate([xp[:, DH:D], xp[:, D + DH:]], axis=1)
    x = jnp.stack([x_even, x_odd], axis=1).reshape(2 * nb, D)
    mean = jnp.sum(s1_ref[...], axis=0, keepdims=True) / E_EDGES
    var = jnp.sum(s2_ref[...], axis=0, keepdims=True) / E_EDGES - mean * mean
    xn = g_ref[...] * (x - mean) * jax.lax.rsqrt(var + 1e-5) + b_ref[...]
    f = jnp.maximum(jnp.dot(xn, w1_ref[...], preferred_element_type=jnp.float32) + b1_ref[...], 0.0)
    o_ref[...] = x + jnp.dot(f, w2_ref[...], preferred_element_type=jnp.float32) + b2_ref[...]


def _sc_edge_body(src_hbm, dst_hbm, cb_hbm, dh_hbm, ei_hbm,
                  e2_hbm, acc_hbm, st_hbm,
                  src_v, dst_v, sdst_v, cb_v, dh_v, ei_v, stage_v, e2p_v, st_v,
                  acc_sh,
                  ssem0, ssem1, dsem0, dsem1,
                  csem0, csem1, hsem0, hsem1, esem0, esem1, osem):
    c = lax.axis_index("c")
    s = lax.axis_index("s")
    c0 = pl.multiple_of(c * DH, DH)
    ssem = (ssem0, ssem1)
    dsem = (dsem0, dsem1)
    csem = (csem0, csem1)
    hsem = (hsem0, hsem1)
    esem = (esem0, esem1)

    def _zst(i, _):
        for j in range(D // L):
            st_v[i, pl.ds(j * L, L)] = jnp.zeros((L,), jnp.float32)
        return 0
    lax.fori_loop(0, 8, _zst, 0)

    def _zrow(i, _):
        for j in range(D // L):
            stage_v[i, pl.ds(j * L, L)] = jnp.zeros((L,), jnp.float32)
        return 0
    lax.fori_loop(0, CH, _zrow, 0)

    def _zcopy(k, _):
        pltpu.sync_copy(stage_v, acc_sh.at[pl.ds(s * PERT + k * CH, CH)])
        return 0
    lax.fori_loop(0, PERT // CH, _zcopy, 0)
    plsc.subcore_barrier()

    off = c * N_NODES

    def _gbase(i):
        return pl.multiple_of((i * NS + s) * CH, 8)

    def _issue_idx(i, b):
        base = _gbase(i)
        pltpu.async_copy(src_hbm.at[pl.ds(base, CH)], src_v.at[b], ssem[b])
        pltpu.async_copy(dst_hbm.at[pl.ds(base, CH)], dst_v.at[b], dsem[b])

    def _wait_idx(i, b):
        base = _gbase(i)
        pltpu.make_async_copy(src_hbm.at[pl.ds(base, CH)], src_v.at[b], ssem[b]).wait()
        pltpu.make_async_copy(dst_hbm.at[pl.ds(base, CH)], dst_v.at[b], dsem[b]).wait()

    def _issue_gathers(i, b):
        def _adj(j, _):
            sl = pl.ds(j * L, L)
            src_v[b, sl] = src_v[b, sl] + off
            return 0
        lax.fori_loop(0, CH // L, _adj, 0)
        base = _gbase(i)
        pltpu.async_copy(cb_hbm.at[src_v.at[b]], cb_v.at[b], csem[b])
        pltpu.async_copy(dh_hbm.at[dst_v.at[b]], dh_v.at[b], hsem[b])
        pltpu.async_copy(ei_hbm.at[c, pl.ds(base, CH)], ei_v.at[b], esem[b])

    def _wait_gathers(i, b):
        base = _gbase(i)
        pltpu.make_async_copy(cb_hbm.at[src_v.at[b]], cb_v.at[b], csem[b]).wait()
        pltpu.make_async_copy(dh_hbm.at[dst_v.at[b]], dh_v.at[b], hsem[b]).wait()
        pltpu.make_async_copy(ei_hbm.at[c, pl.ds(base, CH)], ei_v.at[b], esem[b]).wait()

    def _snap(b):
        def _cp(j, _):
            sl = pl.ds(j * L, L)
            sdst_v[sl] = dst_v[b, sl]
            return 0
        lax.fori_loop(0, CH // L, _cp, 0)

    def _compute(i, b):
        def _pair(q, _):
            for r_par in range(2):
                r = q * 2 + r_par
                e2col = r_par * DH
                for j in range(DH // L):
                    jL = j * L
                    sl = pl.ds(jL, L)
                    t = cb_v[b, r, sl] + dh_v[b, r, pl.ds(c0 + jL, L)] + ei_v[b, r, sl]
                    sg = 1.0 / (1.0 + jnp.exp(-t))
                    e2 = t + ei_v[b, r, pl.ds(DH + jL, L)]
                    e2p_v[q, pl.ds(e2col + jL, L)] = e2
                    stage_v[r, sl] = cb_v[b, r, pl.ds(DH + jL, L)] * sg
                    stage_v[r, pl.ds(DH + jL, L)] = sg
                    plsc.addupdate(st_v.at[0, pl.ds(c0 + jL, L)], e2)
                    plsc.addupdate(st_v.at[1, pl.ds(c0 + jL, L)], e2 * e2)
            return 0
        lax.fori_loop(0, CH // 2, _pair, 0)

        base = _gbase(i)
        cpo = pltpu.async_copy(
            e2p_v, e2_hbm.at[pl.ds(pl.multiple_of(base // 2, 8), CH // 2),
                             pl.ds(c * D, D)], osem)
        pltpu.sync_copy(stage_v, acc_sh.at[sdst_v], add=True)
        cpo.wait()

    _issue_idx(0, 0)
    _wait_idx(0, 0)
    _issue_gathers(0, 0)
    _issue_idx(1, 1)

    def _two(i2, _):
        for b in (1, 0):
            i = 2 * i2 + (1 if b == 1 else 2)
            _wait_idx(i, b)
            _issue_gathers(i, b)
            _wait_gathers(i - 1, 1 - b)
            _snap(1 - b)

            @pl.when(i + 1 < KK)
            def _():
                _issue_idx(i + 1, 1 - b)
            _compute(i - 1, 1 - b)
        return 0

    lax.fori_loop(0, (KK - 1) // 2, _two, 0)

    lastb = (KK - 1) % 2
    _wait_gathers(KK - 1, lastb)
    _snap(lastb)
    _compute(KK - 1, lastb)

    pltpu.sync_copy(st_v, st_hbm.at[c * NS + s])

    plsc.subcore_barrier()
    pltpu.sync_copy(acc_sh.at[pl.ds(s * PERT, PERT)],
                    acc_hbm.at[c, pl.ds(s * PERT, PERT)])


@functools.lru_cache(maxsize=None)
def _sc_kernels():
    mesh = plsc.VectorSubcoreMesh(core_axis_name="c", subcore_axis_name="s",
                                  num_cores=NC, num_subcores=NS)
    edge_pass = pl.kernel(
        _sc_edge_body,
        out_type=(
            jax.ShapeDtypeStruct((E_EDGES // 2, 2 * D), jnp.float32),
            jax.ShapeDtypeStruct((NC, NP, D), jnp.float32),
            jax.ShapeDtypeStruct((NC * NS, 8, D), jnp.float32),
        ),
        mesh=mesh,
        scratch_types=[
            pltpu.VMEM((2, CH), jnp.int32),
            pltpu.VMEM((2, CH), jnp.int32),
            pltpu.VMEM((CH,), jnp.int32),
            pltpu.VMEM((2, CH, D), jnp.float32),
            pltpu.VMEM((2, CH, D), jnp.float32),
            pltpu.VMEM((2, CH, D), jnp.float32),
            pltpu.VMEM((CH, D), jnp.float32),
            pltpu.VMEM((CH // 2, D), jnp.float32),
            pltpu.VMEM((8, D), jnp.float32),
            pltpu.VMEM_SHARED((NP, D), jnp.float32),
        ] + [pltpu.SemaphoreType.DMA] * 11,
    )
    return edge_pass


def kernel(h, e, edge_index, W_A, b_A, W_B, b_B, W_C, b_C, W_D, b_D, W_E, b_E,
           ffh_W1, ffh_b1, ffh_W2, ffh_b2, ffe_W1, ffe_b1, ffe_W2, ffe_b2,
           g1h_g, g1h_b, g1e_g, g1e_b, g2h_g, g2h_b, g2e_g, g2e_b):
    src = edge_index[0]
    dst = edge_index[1]
    row = lambda v: v.reshape(1, D)

    ah, bh, ch, dh = pl.pallas_call(
        _h_pre_body,
        out_shape=[jax.ShapeDtypeStruct((N_NODES, D), jnp.float32)] * 4,
    )(h, row(g1h_g), row(g1h_b), W_A, row(b_A), W_B, row(b_B),
      W_C, row(b_C), W_D, row(b_D))

    BLK = 2000
    grid = E_EDGES // BLK
    estats = pl.pallas_call(
        _colstats_body,
        grid=(grid,),
        in_specs=[pl.BlockSpec((BLK, D), lambda i: (i, 0))],
        out_specs=pl.BlockSpec((8, D), lambda i: (0, 0)),
        out_shape=jax.ShapeDtypeStruct((8, D), jnp.float32),
    )(e)

    ei = pl.pallas_call(
        _e_pre_body,
        grid=(grid,),
        in_specs=[
            pl.BlockSpec((BLK, D), lambda i: (i, 0)),
            pl.BlockSpec((8, D), lambda i: (0, 0)),
            pl.BlockSpec((1, D), lambda i: (0, 0)),
            pl.BlockSpec((1, D), lambda i: (0, 0)),
            pl.BlockSpec((D, D), lambda i: (0, 0)),
            pl.BlockSpec((1, D), lambda i: (0, 0)),
        ],
        out_specs=pl.BlockSpec((NC, BLK, D), lambda i: (0, i, 0)),
        out_shape=jax.ShapeDtypeStruct((NC, E_EDGES, D), jnp.float32),
    )(e, estats, row(g1e_g), row(g1e_b), W_E, row(b_E))

    cb = jnp.concatenate([jnp.concatenate([ch[:, :DH], bh[:, :DH]], axis=1),
                          jnp.concatenate([ch[:, DH:], bh[:, DH:]], axis=1)], axis=0)

    sc_edge = _sc_kernels()
    e2p, acc, st = sc_edge(src, dst, cb, dh, ei)

    acc_h = jnp.concatenate([acc[0, :N_NODES, :DH], acc[1, :N_NODES, :DH]], axis=1)
    acc_s = jnp.concatenate([acc[0, :N_NODES, DH:], acc[1, :N_NODES, DH:]], axis=1)

    h_out = pl.pallas_call(
        _h_post_body,
        out_shape=jax.ShapeDtypeStruct((N_NODES, D), jnp.float32),
    )(ah, acc_h, acc_s, h, row(g2h_g), row(g2h_b),
      ffh_W1, row(ffh_b1), ffh_W2, row(ffh_b2))

    e_out = pl.pallas_call(
        _e_post_body,
        grid=(grid,),
        in_specs=[
            pl.BlockSpec((BLK // 2, 2 * D), lambda i: (i, 0)),
            pl.BlockSpec((NC * NS, D), lambda i: (0, 0)),
            pl.BlockSpec((NC * NS, D), lambda i: (0, 0)),
            pl.BlockSpec((1, D), lambda i: (0, 0)),
            pl.BlockSpec((1, D), lambda i: (0, 0)),
            pl.BlockSpec((D, D), lambda i: (0, 0)),
            pl.BlockSpec((1, D), lambda i: (0, 0)),
            pl.BlockSpec((D, D), lambda i: (0, 0)),
            pl.BlockSpec((1, D), lambda i: (0, 0)),
        ],
        out_specs=pl.BlockSpec((BLK, D), lambda i: (i, 0)),
        out_shape=jax.ShapeDtypeStruct((E_EDGES, D), jnp.float32),
    )(e2p, st[:, 0, :], st[:, 1, :], row(g2e_g), row(g2e_b),
      ffe_W1, row(ffe_b1), ffe_W2, row(ffe_b2))

    return (h_out, e_out)


# speedup vs baseline: 1.3158x; 1.0499x over previous
import functools

import jax
import jax.numpy as jnp
from jax import lax
from jax.experimental import pallas as pl
from jax.experimental.pallas import tpu as pltpu
from jax.experimental.pallas import tpu_sc as plsc

N_NODES = 10000
E_EDGES = 320000
D = 128
DH = D // 2
NC = 2
NS = 16
L = 16

CH = 32
NCHT = E_EDGES // CH
KK = NCHT // NS

NP = 10240
PERT = NP // NS


def _h_pre_body(h_ref, g_ref, b_ref, wa_ref, ba_ref, wb_ref, bb_ref,
                wc_ref, bc_ref, wd_ref, bd_ref,
                ah_ref, bh_ref, ch_ref, dh_ref):
    x = h_ref[...]
    mean = jnp.mean(x, axis=0, keepdims=True)
    xc = x - mean
    var = jnp.mean(xc * xc, axis=0, keepdims=True)
    xn = g_ref[...] * xc * jax.lax.rsqrt(var + 1e-5) + b_ref[...]
    ah_ref[...] = jnp.dot(xn, wa_ref[...], preferred_element_type=jnp.float32) + ba_ref[...]
    bh_ref[...] = jnp.dot(xn, wb_ref[...], preferred_element_type=jnp.float32) + bb_ref[...]
    ch_ref[...] = jnp.dot(xn, wc_ref[...], preferred_element_type=jnp.float32) + bc_ref[...]
    dh_ref[...] = jnp.dot(xn, wd_ref[...], preferred_element_type=jnp.float32) + bd_ref[...]


def _colstats_body(x_ref, o_ref):
    i = pl.program_id(0)
    x = x_ref[...]
    s1 = jnp.sum(x, axis=0, keepdims=True)
    s2 = jnp.sum(x * x, axis=0, keepdims=True)
    blk = jnp.concatenate([s1, s2, jnp.zeros((6, D), jnp.float32)], axis=0)

    @pl.when(i == 0)
    def _init():
        o_ref[...] = blk

    @pl.when(i != 0)
    def _acc():
        o_ref[...] += blk


def _e_pre_body(x_ref, st_ref, g_ref, b_ref, we_ref, be_ref, o_ref):
    x = x_ref[...]
    mean = st_ref[0:1, :] / E_EDGES
    var = st_ref[1:2, :] / E_EDGES - mean * mean
    xn = g_ref[...] * (x - mean) * jax.lax.rsqrt(var + 1e-5) + b_ref[...]
    ee = jnp.dot(xn, we_ref[...], preferred_element_type=jnp.float32) + be_ref[...]
    o_ref[0] = jnp.concatenate([ee[:, :DH], x[:, :DH]], axis=1)
    o_ref[1] = jnp.concatenate([ee[:, DH:], x[:, DH:]], axis=1)


def _h_post_body(ah_ref, acch_ref, accs_ref, hin_ref, g_ref, b_ref,
                 w1_ref, b1_ref, w2_ref, b2_ref, o_ref):
    hmid = ah_ref[...] + acch_ref[...] / (accs_ref[...] + 1e-10)
    h2 = hin_ref[...] + hmid
    mean = jnp.mean(h2, axis=0, keepdims=True)
    xc = h2 - mean
    var = jnp.mean(xc * xc, axis=0, keepdims=True)
    xn = g_ref[...] * xc * jax.lax.rsqrt(var + 1e-5) + b_ref[...]
    f = jnp.maximum(jnp.dot(xn, w1_ref[...], preferred_element_type=jnp.float32) + b1_ref[...], 0.0)
    o_ref[...] = h2 + jnp.dot(f, w2_ref[...], preferred_element_type=jnp.float32) + b2_ref[...]


def _e_post_body(xp_ref, s1_ref, s2_ref, g_ref, b_ref, w1_ref, b1_ref, w2_ref, b2_ref, o_ref):
    xp = xp_ref[...]
    nb = xp.shape[0]
    x_even = jnp.concatenate([xp[:, 0:DH], xp[:, D:D + DH]], axis=1)
    x_odd = jnp.concatenate([xp[:, DH:D], xp[:, D + DH:]], axis=1)
    x = jnp.stack([x_even, x_odd], axis=1).reshape(2 * nb, D)
    mean = jnp.sum(s1_ref[...], axis=0, keepdims=True) / E_EDGES
    var = jnp.sum(s2_ref[...], axis=0, keepdims=True) / E_EDGES - mean * mean
    xn = g_ref[...] * (x - mean) * jax.lax.rsqrt(var + 1e-5) + b_ref[...]
    f = jnp.maximum(jnp.dot(xn, w1_ref[...], preferred_element_type=jnp.float32) + b1_ref[...], 0.0)
    o_ref[...] = x + jnp.dot(f, w2_ref[...], preferred_element_type=jnp.float32) + b2_ref[...]


def _sc_edge_body(src_hbm, dst_hbm, cb_hbm, dh_hbm, ei_hbm,
                  e2_hbm, acc_hbm, st_hbm,
                  src_v, dst_v, sdst_v, cb_v, dh_v, ei_v, stage_v, e2p_v, st_v,
                  acc_sh,
                  ssem0, ssem1, dsem0, dsem1,
                  csem0, csem1, hsem0, hsem1, esem0, esem1,
                  gsem0, gsem1, osem0, osem1):
    c = lax.axis_index("c")
    s = lax.axis_index("s")
    c0 = pl.multiple_of(c * DH, DH)
    ssem = (ssem0, ssem1)
    dsem = (dsem0, dsem1)
    csem = (csem0, csem1)
    hsem = (hsem0, hsem1)
    esem = (esem0, esem1)
    gsem = (gsem0, gsem1)
    osem = (osem0, osem1)

    def _zst(i, _):
        for j in range(D // L):
            st_v[i, pl.ds(j * L, L)] = jnp.zeros((L,), jnp.float32)
        return 0
    lax.fori_loop(0, 8, _zst, 0)

    def _zrow(i, _):
        for j in range(D // L):
            stage_v[0, i, pl.ds(j * L, L)] = jnp.zeros((L,), jnp.float32)
        return 0
    lax.fori_loop(0, CH, _zrow, 0)

    def _zcopy(k, _):
        pltpu.sync_copy(stage_v.at[0], acc_sh.at[pl.ds(s * PERT + k * CH, CH)])
        return 0
    lax.fori_loop(0, PERT // CH, _zcopy, 0)
    plsc.subcore_barrier()

    off = c * N_NODES

    def _gbase(i):
        return pl.multiple_of((i * NS + s) * CH, 8)

    def _issue_idx(i, b):
        base = _gbase(i)
        pltpu.async_copy(src_hbm.at[pl.ds(base, CH)], src_v.at[b], ssem[b])
        pltpu.async_copy(dst_hbm.at[pl.ds(base, CH)], dst_v.at[b], dsem[b])

    def _wait_idx(i, b):
        base = _gbase(i)
        pltpu.make_async_copy(src_hbm.at[pl.ds(base, CH)], src_v.at[b], ssem[b]).wait()
        pltpu.make_async_copy(dst_hbm.at[pl.ds(base, CH)], dst_v.at[b], dsem[b]).wait()

    def _issue_gathers(i, b):
        def _adj(j, _):
            sl = pl.ds(j * L, L)
            src_v[b, sl] = src_v[b, sl] + off
            return 0
        lax.fori_loop(0, CH // L, _adj, 0)
        base = _gbase(i)
        pltpu.async_copy(cb_hbm.at[src_v.at[b]], cb_v.at[b], csem[b])
        pltpu.async_copy(dh_hbm.at[dst_v.at[b]], dh_v.at[b], hsem[b])
        pltpu.async_copy(ei_hbm.at[c, pl.ds(base, CH)], ei_v.at[b], esem[b])

    def _wait_gathers(i, b):
        base = _gbase(i)
        pltpu.make_async_copy(cb_hbm.at[src_v.at[b]], cb_v.at[b], csem[b]).wait()
        pltpu.make_async_copy(dh_hbm.at[dst_v.at[b]], dh_v.at[b], hsem[b]).wait()
        pltpu.make_async_copy(ei_hbm.at[c, pl.ds(base, CH)], ei_v.at[b], esem[b]).wait()

    def _snap(b):
        def _cp(j, _):
            sl = pl.ds(j * L, L)
            sdst_v[b, sl] = dst_v[b, sl]
            return 0
        lax.fori_loop(0, CH // L, _cp, 0)

    def _e2dst(i):
        base = _gbase(i)
        return e2_hbm.at[pl.ds(pl.multiple_of(base // 2, 8), CH // 2),
                         pl.ds(c * D, D)]

    def _wait_outs(i, b):
        @pl.when(i >= 2)
        def _():
            pltpu.make_async_copy(stage_v.at[b], acc_sh.at[sdst_v.at[b]],
                                  gsem[b]).wait()
            pltpu.make_async_copy(e2p_v.at[b], _e2dst(i), osem[b]).wait()

    def _compute(i, b):
        def _pair(q, _):
            for r_par in range(2):
                r = q * 2 + r_par
                e2col = r_par * DH
                for j in range(DH // L):
                    jL = j * L
                    sl = pl.ds(jL, L)
                    t = cb_v[b, r, sl] + dh_v[b, r, pl.ds(c0 + jL, L)] + ei_v[b, r, sl]
                    sg = 1.0 / (1.0 + jnp.exp(-t))
                    e2 = t + ei_v[b, r, pl.ds(DH + jL, L)]
                    e2p_v[b, q, pl.ds(e2col + jL, L)] = e2
                    stage_v[b, r, sl] = cb_v[b, r, pl.ds(DH + jL, L)] * sg
                    stage_v[b, r, pl.ds(DH + jL, L)] = sg
                    plsc.addupdate(st_v.at[0, pl.ds(c0 + jL, L)], e2)
                    plsc.addupdate(st_v.at[1, pl.ds(c0 + jL, L)], e2 * e2)
            return 0
        lax.fori_loop(0, CH // 2, _pair, 0)

        pltpu.async_copy(e2p_v.at[b], _e2dst(i), osem[b])
        pltpu.async_copy(stage_v.at[b], acc_sh.at[sdst_v.at[b]], gsem[b],
                         add=True)

    _issue_idx(0, 0)
    _wait_idx(0, 0)
    _issue_gathers(0, 0)
    _issue_idx(1, 1)

    def _two(i2, _):
        for b in (1, 0):
            i = 2 * i2 + (1 if b == 1 else 2)
            _wait_idx(i, b)
            _issue_gathers(i, b)
            _wait_gathers(i - 1, 1 - b)
            _wait_outs(i - 1, 1 - b)
            _snap(1 - b)

            @pl.when(i + 1 < KK)
            def _():
                _issue_idx(i + 1, 1 - b)
            _compute(i - 1, 1 - b)
        return 0

    lax.fori_loop(0, (KK - 1) // 2, _two, 0)

    lastb = (KK - 1) % 2
    _wait_gathers(KK - 1, lastb)
    _wait_outs(KK - 1, lastb)
    _snap(lastb)
    _compute(KK - 1, lastb)

    for b in (0, 1):
        pltpu.make_async_copy(stage_v.at[b], acc_sh.at[sdst_v.at[b]],
                              gsem[b]).wait()
        pltpu.make_async_copy(e2p_v.at[b], _e2dst(KK - 1), osem[b]).wait()

    pltpu.sync_copy(st_v, st_hbm.at[c * NS + s])

    plsc.subcore_barrier()
    pltpu.sync_copy(acc_sh.at[pl.ds(s * PERT, PERT)],
                    acc_hbm.at[c, pl.ds(s * PERT, PERT)])


@functools.lru_cache(maxsize=None)
def _sc_kernels():
    mesh = plsc.VectorSubcoreMesh(core_axis_name="c", subcore_axis_name="s",
                                  num_cores=NC, num_subcores=NS)
    edge_pass = pl.kernel(
        _sc_edge_body,
        out_type=(
            jax.ShapeDtypeStruct((E_EDGES // 2, 2 * D), jnp.float32),
            jax.ShapeDtypeStruct((NC, NP, D), jnp.float32),
            jax.ShapeDtypeStruct((NC * NS, 8, D), jnp.float32),
        ),
        mesh=mesh,
        scratch_types=[
            pltpu.VMEM((2, CH), jnp.int32),
            pltpu.VMEM((2, CH), jnp.int32),
            pltpu.VMEM((2, CH), jnp.int32),
            pltpu.VMEM((2, CH, D), jnp.float32),
            pltpu.VMEM((2, CH, D), jnp.float32),
            pltpu.VMEM((2, CH, D), jnp.float32),
            pltpu.VMEM((2, CH, D), jnp.float32),
            pltpu.VMEM((2, CH // 2, D), jnp.float32),
            pltpu.VMEM((8, D), jnp.float32),
            pltpu.VMEM_SHARED((NP, D), jnp.float32),
        ] + [pltpu.SemaphoreType.DMA] * 14,
    )
    return edge_pass


def kernel(h, e, edge_index, W_A, b_A, W_B, b_B, W_C, b_C, W_D, b_D, W_E, b_E,
           ffh_W1, ffh_b1, ffh_W2, ffh_b2, ffe_W1, ffe_b1, ffe_W2, ffe_b2,
           g1h_g, g1h_b, g1e_g, g1e_b, g2h_g, g2h_b, g2e_g, g2e_b):
    src = edge_index[0]
    dst = edge_index[1]
    row = lambda v: v.reshape(1, D)

    ah, bh, ch, dh = pl.pallas_call(
        _h_pre_body,
        out_shape=[jax.ShapeDtypeStruct((N_NODES, D), jnp.float32)] * 4,
    )(h, row(g1h_g), row(g1h_b), W_A, row(b_A), W_B, row(b_B),
      W_C, row(b_C), W_D, row(b_D))

    BLK = 2000
    grid = E_EDGES // BLK
    estats = pl.pallas_call(
        _colstats_body,
        grid=(grid,),
        in_specs=[pl.BlockSpec((BLK, D), lambda i: (i, 0))],
        out_specs=pl.BlockSpec((8, D), lambda i: (0, 0)),
        out_shape=jax.ShapeDtypeStruct((8, D), jnp.float32),
    )(e)

    ei = pl.pallas_call(
        _e_pre_body,
        grid=(grid,),
        in_specs=[
            pl.BlockSpec((BLK, D), lambda i: (i, 0)),
            pl.BlockSpec((8, D), lambda i: (0, 0)),
            pl.BlockSpec((1, D), lambda i: (0, 0)),
            pl.BlockSpec((1, D), lambda i: (0, 0)),
            pl.BlockSpec((D, D), lambda i: (0, 0)),
            pl.BlockSpec((1, D), lambda i: (0, 0)),
        ],
        out_specs=pl.BlockSpec((NC, BLK, D), lambda i: (0, i, 0)),
        out_shape=jax.ShapeDtypeStruct((NC, E_EDGES, D), jnp.float32),
    )(e, estats, row(g1e_g), row(g1e_b), W_E, row(b_E))

    cb = jnp.concatenate([jnp.concatenate([ch[:, :DH], bh[:, :DH]], axis=1),
                          jnp.concatenate([ch[:, DH:], bh[:, DH:]], axis=1)], axis=0)

    sc_edge = _sc_kernels()
    e2p, acc, st = sc_edge(src, dst, cb, dh, ei)

    acc_h = jnp.concatenate([acc[0, :N_NODES, :DH], acc[1, :N_NODES, :DH]], axis=1)
    acc_s = jnp.concatenate([acc[0, :N_NODES, DH:], acc[1, :N_NODES, DH:]], axis=1)

    h_out = pl.pallas_call(
        _h_post_body,
        out_shape=jax.ShapeDtypeStruct((N_NODES, D), jnp.float32),
    )(ah, acc_h, acc_s, h, row(g2h_g), row(g2h_b),
      ffh_W1, row(ffh_b1), ffh_W2, row(ffh_b2))

    e_out = pl.pallas_call(
        _e_post_body,
        grid=(grid,),
        in_specs=[
            pl.BlockSpec((BLK // 2, 2 * D), lambda i: (i, 0)),
            pl.BlockSpec((NC * NS, D), lambda i: (0, 0)),
            pl.BlockSpec((NC * NS, D), lambda i: (0, 0)),
            pl.BlockSpec((1, D), lambda i: (0, 0)),
            pl.BlockSpec((1, D), lambda i: (0, 0)),
            pl.BlockSpec((D, D), lambda i: (0, 0)),
            pl.BlockSpec((1, D), lambda i: (0, 0)),
            pl.BlockSpec((D, D), lambda i: (0, 0)),
            pl.BlockSpec((1, D), lambda i: (0, 0)),
        ],
        out_specs=pl.BlockSpec((BLK, D), lambda i: (i, 0)),
        out_shape=jax.ShapeDtypeStruct((E_EDGES, D), jnp.float32),
    )(e2p, st[:, 0, :], st[:, 1, :], row(g2e_g), row(g2e_b),
      ffe_W1, row(ffe_b1), ffe_W2, row(ffe_b2))

    return (h_out, e_out)


# speedup vs baseline: 1.3840x; 1.0519x over previous
import functools

import jax
import jax.numpy as jnp
from jax import lax
from jax.experimental import pallas as pl
from jax.experimental.pallas import tpu as pltpu
from jax.experimental.pallas import tpu_sc as plsc

N_NODES = 10000
E_EDGES = 320000
D = 128
DH = D // 2
NC = 2
NS = 16
L = 16

CH = 32
NCHT = E_EDGES // CH
KK = NCHT // NS

NP = 10240
PERT = NP // NS


def _h_pre_body(h_ref, g_ref, b_ref, wa_ref, ba_ref, wb_ref, bb_ref,
                wc_ref, bc_ref, wd_ref, bd_ref,
                ah_ref, bh_ref, ch_ref, dh_ref):
    x = h_ref[...]
    mean = jnp.mean(x, axis=0, keepdims=True)
    xc = x - mean
    var = jnp.mean(xc * xc, axis=0, keepdims=True)
    xn = g_ref[...] * xc * jax.lax.rsqrt(var + 1e-5) + b_ref[...]
    ah_ref[...] = jnp.dot(xn, wa_ref[...], preferred_element_type=jnp.float32) + ba_ref[...]
    bh_ref[...] = jnp.dot(xn, wb_ref[...], preferred_element_type=jnp.float32) + bb_ref[...]
    ch_ref[...] = jnp.dot(xn, wc_ref[...], preferred_element_type=jnp.float32) + bc_ref[...]
    dh_ref[...] = jnp.dot(xn, wd_ref[...], preferred_element_type=jnp.float32) + bd_ref[...]


def _colstats_body(x_ref, o_ref):
    i = pl.program_id(0)
    x = x_ref[...]
    s1 = jnp.sum(x, axis=0, keepdims=True)
    s2 = jnp.sum(x * x, axis=0, keepdims=True)
    blk = jnp.concatenate([s1, s2, jnp.zeros((6, D), jnp.float32)], axis=0)

    @pl.when(i == 0)
    def _init():
        o_ref[...] = blk

    @pl.when(i != 0)
    def _acc():
        o_ref[...] += blk


def _e_pre_body(x_ref, st_ref, g_ref, b_ref, we_ref, be_ref, o_ref):
    x = x_ref[...]
    mean = st_ref[0:1, :] / E_EDGES
    var = st_ref[1:2, :] / E_EDGES - mean * mean
    xn = g_ref[...] * (x - mean) * jax.lax.rsqrt(var + 1e-5) + b_ref[...]
    ee = jnp.dot(xn, we_ref[...], preferred_element_type=jnp.float32) + be_ref[...]
    o_ref[0] = jnp.concatenate([ee[:, :DH], x[:, :DH]], axis=1)
    o_ref[1] = jnp.concatenate([ee[:, DH:], x[:, DH:]], axis=1)


def _h_post_body(ah_ref, acch_ref, accs_ref, hin_ref, g_ref, b_ref,
                 w1_ref, b1_ref, w2_ref, b2_ref, o_ref):
    hmid = ah_ref[...] + acch_ref[...] / (accs_ref[...] + 1e-10)
    h2 = hin_ref[...] + hmid
    mean = jnp.mean(h2, axis=0, keepdims=True)
    xc = h2 - mean
    var = jnp.mean(xc * xc, axis=0, keepdims=True)
    xn = g_ref[...] * xc * jax.lax.rsqrt(var + 1e-5) + b_ref[...]
    f = jnp.maximum(jnp.dot(xn, w1_ref[...], preferred_element_type=jnp.float32) + b1_ref[...], 0.0)
    o_ref[...] = h2 + jnp.dot(f, w2_ref[...], preferred_element_type=jnp.float32) + b2_ref[...]


def _e_post_body(xp_ref, s1_ref, s2_ref, g_ref, b_ref, w1_ref, b1_ref, w2_ref, b2_ref, o_ref):
    xp = xp_ref[...]
    nb = xp.shape[0]
    x_even = jnp.concatenate([xp[:, 0:DH], xp[:, D:D + DH]], axis=1)
    x_odd = jnp.concatenate([xp[:, DH:D], xp[:, D + DH:]], axis=1)
    x = jnp.stack([x_even, x_odd], axis=1).reshape(2 * nb, D)
    mean = jnp.sum(s1_ref[...], axis=0, keepdims=True) / E_EDGES
    var = jnp.sum(s2_ref[...], axis=0, keepdims=True) / E_EDGES - mean * mean
    xn = g_ref[...] * (x - mean) * jax.lax.rsqrt(var + 1e-5) + b_ref[...]
    f = jnp.maximum(jnp.dot(xn, w1_ref[...], preferred_element_type=jnp.float32) + b1_ref[...], 0.0)
    o_ref[...] = x + jnp.dot(f, w2_ref[...], preferred_element_type=jnp.float32) + b2_ref[...]


def _sc_edge_body(src_hbm, dst_hbm, cb_hbm, dh_hbm, ei_hbm,
                  e2_hbm, acc_hbm, st_hbm,
                  src_v, dst_v, sdst_v, dat_v, stage_v, e2p_v, st_v,
                  acc_sh,
                  ssem0, ssem1, dsem0, dsem1,
                  csem0, csem1, gsem0, gsem1, osem0, osem1):
    c = lax.axis_index("c")
    s = lax.axis_index("s")
    c0 = pl.multiple_of(c * DH, DH)
    ssem = (ssem0, ssem1)
    dsem = (dsem0, dsem1)
    csem = (csem0, csem1)
    gsem = (gsem0, gsem1)
    osem = (osem0, osem1)

    def _zst(i, _):
        for j in range(D // L):
            st_v[i, pl.ds(j * L, L)] = jnp.zeros((L,), jnp.float32)
        return 0
    lax.fori_loop(0, 8, _zst, 0)

    def _zrow(i, _):
        for j in range(D // L):
            stage_v[0, i, pl.ds(j * L, L)] = jnp.zeros((L,), jnp.float32)
        return 0
    lax.fori_loop(0, CH, _zrow, 0)

    def _zcopy(k, _):
        pltpu.sync_copy(stage_v.at[0], acc_sh.at[pl.ds(s * PERT + k * CH, CH)])
        return 0
    lax.fori_loop(0, PERT // CH, _zcopy, 0)
    plsc.subcore_barrier()

    off = c * N_NODES

    def _gbase(i):
        return pl.multiple_of((i * NS + s) * CH, 8)

    def _issue_idx(i, b):
        base = _gbase(i)
        pltpu.async_copy(src_hbm.at[pl.ds(base, CH)], src_v.at[b], ssem[b])
        pltpu.async_copy(dst_hbm.at[pl.ds(base, CH)], dst_v.at[b], dsem[b])

    def _wait_idx(i, b):
        base = _gbase(i)
        pltpu.make_async_copy(src_hbm.at[pl.ds(base, CH)], src_v.at[b], ssem[b]).wait()
        pltpu.make_async_copy(dst_hbm.at[pl.ds(base, CH)], dst_v.at[b], dsem[b]).wait()

    def _issue_gathers(i, b):
        def _adj(j, _):
            sl = pl.ds(j * L, L)
            src_v[b, sl] = src_v[b, sl] + off
            return 0
        lax.fori_loop(0, CH // L, _adj, 0)
        base = _gbase(i)
        pltpu.async_copy(cb_hbm.at[src_v.at[b]], dat_v.at[b, pl.ds(0, CH)], csem[b])
        pltpu.async_copy(dh_hbm.at[dst_v.at[b]], dat_v.at[b, pl.ds(CH, CH)], csem[b])
        pltpu.async_copy(ei_hbm.at[c, pl.ds(base, CH)], dat_v.at[b, pl.ds(2 * CH, CH)], csem[b])

    def _wait_gathers(i, b):
        pltpu.make_async_copy(ei_hbm.at[c, pl.ds(0, 3 * CH)], dat_v.at[b],
                              csem[b]).wait()

    def _snap(b):
        def _cp(j, _):
            sl = pl.ds(j * L, L)
            sdst_v[b, sl] = dst_v[b, sl]
            return 0
        lax.fori_loop(0, CH // L, _cp, 0)

    def _e2dst(i):
        base = _gbase(i)
        return e2_hbm.at[pl.ds(pl.multiple_of(base // 2, 8), CH // 2),
                         pl.ds(c * D, D)]

    def _wait_outs(i, b):
        @pl.when(i >= 2)
        def _():
            pltpu.make_async_copy(stage_v.at[b], acc_sh.at[sdst_v.at[b]],
                                  gsem[b]).wait()
            pltpu.make_async_copy(e2p_v.at[b], _e2dst(i), osem[b]).wait()

    NJ = DH // L

    def _compute(i, b):
        init = tuple(jnp.zeros((L,), jnp.float32) for _ in range(2 * NJ))

        def _pair_body(q, cr):
            acc = list(cr)
            for r_par in range(2):
                r = q * 2 + r_par
                e2col = r_par * DH
                for j in range(NJ):
                    jL = j * L
                    sl = pl.ds(jL, L)
                    t = dat_v[b, r, sl] + dat_v[b, CH + r, pl.ds(c0 + jL, L)] \
                        + dat_v[b, 2 * CH + r, sl]
                    sg = 1.0 / (1.0 + jnp.exp(-t))
                    e2 = t + dat_v[b, 2 * CH + r, pl.ds(DH + jL, L)]
                    e2p_v[b, q, pl.ds(e2col + jL, L)] = e2
                    stage_v[b, r, sl] = dat_v[b, r, pl.ds(DH + jL, L)] * sg
                    stage_v[b, r, pl.ds(DH + jL, L)] = sg
                    acc[j] = acc[j] + e2
                    acc[NJ + j] = acc[NJ + j] + e2 * e2
            return tuple(acc)

        fin = plsc.parallel_loop(0, CH // 2, unroll=2, carry=init)(_pair_body)
        for j in range(NJ):
            plsc.addupdate(st_v.at[0, pl.ds(c0 + j * L, L)], fin[j])
            plsc.addupdate(st_v.at[1, pl.ds(c0 + j * L, L)], fin[NJ + j])

        pltpu.async_copy(e2p_v.at[b], _e2dst(i), osem[b])
        pltpu.async_copy(stage_v.at[b], acc_sh.at[sdst_v.at[b]], gsem[b],
                         add=True)

    _issue_idx(0, 0)
    _wait_idx(0, 0)
    _issue_gathers(0, 0)
    _issue_idx(1, 1)

    def _two(i2, _):
        for b in (1, 0):
            i = 2 * i2 + (1 if b == 1 else 2)
            _wait_idx(i, b)
            _issue_gathers(i, b)
            _wait_gathers(i - 1, 1 - b)
            _wait_outs(i - 1, 1 - b)
            _snap(1 - b)

            @pl.when(i + 1 < KK)
            def _():
                _issue_idx(i + 1, 1 - b)
            _compute(i - 1, 1 - b)
        return 0

    lax.fori_loop(0, (KK - 1) // 2, _two, 0)

    lastb = (KK - 1) % 2
    _wait_gathers(KK - 1, lastb)
    _wait_outs(KK - 1, lastb)
    _snap(lastb)
    _compute(KK - 1, lastb)

    for b in (0, 1):
        pltpu.make_async_copy(stage_v.at[b], acc_sh.at[sdst_v.at[b]],
                              gsem[b]).wait()
        pltpu.make_async_copy(e2p_v.at[b], _e2dst(KK - 1), osem[b]).wait()

    pltpu.sync_copy(st_v, st_hbm.at[c * NS + s])

    plsc.subcore_barrier()
    pltpu.sync_copy(acc_sh.at[pl.ds(s * PERT, PERT)],
                    acc_hbm.at[c, pl.ds(s * PERT, PERT)])


@functools.lru_cache(maxsize=None)
def _sc_kernels():
    mesh = plsc.VectorSubcoreMesh(core_axis_name="c", subcore_axis_name="s",
                                  num_cores=NC, num_subcores=NS)
    edge_pass = pl.kernel(
        _sc_edge_body,
        out_type=(
            jax.ShapeDtypeStruct((E_EDGES // 2, 2 * D), jnp.float32),
            jax.ShapeDtypeStruct((NC, NP, D), jnp.float32),
            jax.ShapeDtypeStruct((NC * NS, 8, D), jnp.float32),
        ),
        mesh=mesh,
        scratch_types=[
            pltpu.VMEM((2, CH), jnp.int32),
            pltpu.VMEM((2, CH), jnp.int32),
            pltpu.VMEM((2, CH), jnp.int32),
            pltpu.VMEM((2, 3 * CH, D), jnp.float32),
            pltpu.VMEM((2, CH, D), jnp.float32),
            pltpu.VMEM((2, CH // 2, D), jnp.float32),
            pltpu.VMEM((8, D), jnp.float32),
            pltpu.VMEM_SHARED((NP, D), jnp.float32),
        ] + [pltpu.SemaphoreType.DMA] * 10,
    )
    return edge_pass


def kernel(h, e, edge_index, W_A, b_A, W_B, b_B, W_C, b_C, W_D, b_D, W_E, b_E,
           ffh_W1, ffh_b1, ffh_W2, ffh_b2, ffe_W1, ffe_b1, ffe_W2, ffe_b2,
           g1h_g, g1h_b, g1e_g, g1e_b, g2h_g, g2h_b, g2e_g, g2e_b):
    src = edge_index[0]
    dst = edge_index[1]
    row = lambda v: v.reshape(1, D)

    ah, bh, ch, dh = pl.pallas_call(
        _h_pre_body,
        out_shape=[jax.ShapeDtypeStruct((N_NODES, D), jnp.float32)] * 4,
    )(h, row(g1h_g), row(g1h_b), W_A, row(b_A), W_B, row(b_B),
      W_C, row(b_C), W_D, row(b_D))

    BLK = 2000
    grid = E_EDGES // BLK
    estats = pl.pallas_call(
        _colstats_body,
        grid=(grid,),
        in_specs=[pl.BlockSpec((BLK, D), lambda i: (i, 0))],
        out_specs=pl.BlockSpec((8, D), lambda i: (0, 0)),
        out_shape=jax.ShapeDtypeStruct((8, D), jnp.float32),
    )(e)

    ei = pl.pallas_call(
        _e_pre_body,
        grid=(grid,),
        in_specs=[
            pl.BlockSpec((BLK, D), lambda i: (i, 0)),
            pl.BlockSpec((8, D), lambda i: (0, 0)),
            pl.BlockSpec((1, D), lambda i: (0, 0)),
            pl.BlockSpec((1, D), lambda i: (0, 0)),
            pl.BlockSpec((D, D), lambda i: (0, 0)),
            pl.BlockSpec((1, D), lambda i: (0, 0)),
        ],
        out_specs=pl.BlockSpec((NC, BLK, D), lambda i: (0, i, 0)),
        out_shape=jax.ShapeDtypeStruct((NC, E_EDGES, D), jnp.float32),
    )(e, estats, row(g1e_g), row(g1e_b), W_E, row(b_E))

    cb = jnp.concatenate([jnp.concatenate([ch[:, :DH], bh[:, :DH]], axis=1),
                          jnp.concatenate([ch[:, DH:], bh[:, DH:]], axis=1)], axis=0)

    sc_edge = _sc_kernels()
    e2p, acc, st = sc_edge(src, dst, cb, dh, ei)

    acc_h = jnp.concatenate([acc[0, :N_NODES, :DH], acc[1, :N_NODES, :DH]], axis=1)
    acc_s = jnp.concatenate([acc[0, :N_NODES, DH:], acc[1, :N_NODES, DH:]], axis=1)

    h_out = pl.pallas_call(
        _h_post_body,
        out_shape=jax.ShapeDtypeStruct((N_NODES, D), jnp.float32),
    )(ah, acc_h, acc_s, h, row(g2h_g), row(g2h_b),
      ffh_W1, row(ffh_b1), ffh_W2, row(ffh_b2))

    e_out = pl.pallas_call(
        _e_post_body,
        grid=(grid,),
        in_specs=[
            pl.BlockSpec((BLK // 2, 2 * D), lambda i: (i, 0)),
            pl.BlockSpec((NC * NS, D), lambda i: (0, 0)),
            pl.BlockSpec((NC * NS, D), lambda i: (0, 0)),
            pl.BlockSpec((1, D), lambda i: (0, 0)),
            pl.BlockSpec((1, D), lambda i: (0, 0)),
            pl.BlockSpec((D, D), lambda i: (0, 0)),
            pl.BlockSpec((1, D), lambda i: (0, 0)),
            pl.BlockSpec((D, D), lambda i: (0, 0)),
            pl.BlockSpec((1, D), lambda i: (0, 0)),
        ],
        out_specs=pl.BlockSpec((BLK, D), lambda i: (i, 0)),
        out_shape=jax.ShapeDtypeStruct((E_EDGES, D), jnp.float32),
    )(e2p, st[:, 0, :], st[:, 1, :], row(g2e_g), row(g2e_b),
      ffe_W1, row(ffe_b1), ffe_W2, row(ffe_b2))

    return (h_out, e_out)


# speedup vs baseline: 2.0097x; 1.4521x over previous
import functools

import jax
import jax.numpy as jnp
from jax import lax
from jax.experimental import pallas as pl
from jax.experimental.pallas import tpu as pltpu
from jax.experimental.pallas import tpu_sc as plsc

N_NODES = 10000
E_EDGES = 320000
D = 128
DH = D // 2
NC = 2
NS = 16
L = 16

CH = 32
NCHT = E_EDGES // CH
KK = NCHT // NS

NP = 10240
PERT = NP // NS


def _h_pre_body(h_ref, g_ref, b_ref, wa_ref, ba_ref, wb_ref, bb_ref,
                wc_ref, bc_ref, wd_ref, bd_ref,
                ah_ref, bh_ref, ch_ref, dh_ref):
    x = h_ref[...]
    mean = jnp.mean(x, axis=0, keepdims=True)
    xc = x - mean
    var = jnp.mean(xc * xc, axis=0, keepdims=True)
    xn = g_ref[...] * xc * jax.lax.rsqrt(var + 1e-5) + b_ref[...]
    ah_ref[...] = jnp.dot(xn, wa_ref[...], preferred_element_type=jnp.float32) + ba_ref[...]
    bh_ref[...] = jnp.dot(xn, wb_ref[...], preferred_element_type=jnp.float32) + bb_ref[...]
    ch_ref[...] = jnp.dot(xn, wc_ref[...], preferred_element_type=jnp.float32) + bc_ref[...]
    dh_ref[...] = jnp.dot(xn, wd_ref[...], preferred_element_type=jnp.float32) + bd_ref[...]


def _colstats_body(x_ref, o_ref):
    i = pl.program_id(0)
    x = x_ref[...]
    s1 = jnp.sum(x, axis=0, keepdims=True)
    s2 = jnp.sum(x * x, axis=0, keepdims=True)
    blk = jnp.concatenate([s1, s2, jnp.zeros((6, D), jnp.float32)], axis=0)

    @pl.when(i == 0)
    def _init():
        o_ref[...] = blk

    @pl.when(i != 0)
    def _acc():
        o_ref[...] += blk


def _e_pre_body(x_ref, st_ref, g_ref, b_ref, we_ref, be_ref, o_ref):
    x = x_ref[...]
    mean = st_ref[0:1, :] / E_EDGES
    var = st_ref[1:2, :] / E_EDGES - mean * mean
    xn = g_ref[...] * (x - mean) * jax.lax.rsqrt(var + 1e-5) + b_ref[...]
    ee = jnp.dot(xn, we_ref[...], preferred_element_type=jnp.float32) + be_ref[...]
    o_ref[0] = jnp.concatenate([ee[:, :DH], x[:, :DH]], axis=1)
    o_ref[1] = jnp.concatenate([ee[:, DH:], x[:, DH:]], axis=1)


def _h_post_body(ah_ref, acch_ref, accs_ref, hin_ref, g_ref, b_ref,
                 w1_ref, b1_ref, w2_ref, b2_ref, o_ref):
    hmid = ah_ref[...] + acch_ref[...] / (accs_ref[...] + 1e-10)
    h2 = hin_ref[...] + hmid
    mean = jnp.mean(h2, axis=0, keepdims=True)
    xc = h2 - mean
    var = jnp.mean(xc * xc, axis=0, keepdims=True)
    xn = g_ref[...] * xc * jax.lax.rsqrt(var + 1e-5) + b_ref[...]
    f = jnp.maximum(jnp.dot(xn, w1_ref[...], preferred_element_type=jnp.float32) + b1_ref[...], 0.0)
    o_ref[...] = h2 + jnp.dot(f, w2_ref[...], preferred_element_type=jnp.float32) + b2_ref[...]


def _e_post_body(xp_ref, s1_ref, s2_ref, g_ref, b_ref, w1_ref, b1_ref, w2_ref, b2_ref, o_ref):
    xp = xp_ref[...]
    nb = xp.shape[0]
    x_even = jnp.concatenate([xp[:, 0:DH], xp[:, D:D + DH]], axis=1)
    x_odd = jnp.concatenate([xp[:, DH:D], xp[:, D + DH:]], axis=1)
    x = jnp.stack([x_even, x_odd], axis=1).reshape(2 * nb, D)
    mean = jnp.sum(s1_ref[...], axis=0, keepdims=True) / E_EDGES
    var = jnp.sum(s2_ref[...], axis=0, keepdims=True) / E_EDGES - mean * mean
    xn = g_ref[...] * (x - mean) * jax.lax.rsqrt(var + 1e-5) + b_ref[...]
    f = jnp.maximum(jnp.dot(xn, w1_ref[...], preferred_element_type=jnp.float32) + b1_ref[...], 0.0)
    o_ref[...] = x + jnp.dot(f, w2_ref[...], preferred_element_type=jnp.float32) + b2_ref[...]


def _sc_edge_body(src_hbm, dst_hbm, tbl_hbm, ei_hbm,
                  e2_hbm, acc_hbm, st_hbm,
                  src_v, dst_v, gidx_v, sdst_v, dat_v, stage_v, e2p_v, st_v,
                  acc_sh,
                  ssem0, ssem1, dsem0, dsem1,
                  csem0, csem1, gsem0, gsem1, osem0, osem1):
    c = lax.axis_index("c")
    s = lax.axis_index("s")
    c0 = pl.multiple_of(c * DH, DH)
    ssem = (ssem0, ssem1)
    dsem = (dsem0, dsem1)
    csem = (csem0, csem1)
    gsem = (gsem0, gsem1)
    osem = (osem0, osem1)

    def _zst(i, _):
        for j in range(D // L):
            st_v[i, pl.ds(j * L, L)] = jnp.zeros((L,), jnp.float32)
        return 0
    lax.fori_loop(0, 8, _zst, 0)

    def _zrow(i, _):
        for j in range(D // L):
            stage_v[0, i, pl.ds(j * L, L)] = jnp.zeros((L,), jnp.float32)
        return 0
    lax.fori_loop(0, CH, _zrow, 0)

    def _zcopy(k, _):
        pltpu.sync_copy(stage_v.at[0], acc_sh.at[pl.ds(s * PERT + k * CH, CH)])
        return 0
    lax.fori_loop(0, PERT // CH, _zcopy, 0)
    plsc.subcore_barrier()

    off = c * N_NODES

    def _gbase(i):
        return pl.multiple_of((i * NS + s) * CH, 8)

    def _issue_idx(i, b):
        base = _gbase(i)
        pltpu.async_copy(src_hbm.at[pl.ds(base, CH)], src_v.at[b], ssem[b])
        pltpu.async_copy(dst_hbm.at[pl.ds(base, CH)], dst_v.at[b], dsem[b])

    def _wait_idx(i, b):
        base = _gbase(i)
        pltpu.make_async_copy(src_hbm.at[pl.ds(base, CH)], src_v.at[b], ssem[b]).wait()
        pltpu.make_async_copy(dst_hbm.at[pl.ds(base, CH)], dst_v.at[b], dsem[b]).wait()

    def _issue_gathers(i, b):
        for j in range(CH // L):
            sl = pl.ds(j * L, L)
            gidx_v[b, sl] = src_v[b, sl] + off
            gidx_v[b, pl.ds(CH + j * L, L)] = dst_v[b, sl] + (2 * N_NODES + off)
        base = _gbase(i)
        pltpu.async_copy(tbl_hbm.at[gidx_v.at[b]], dat_v.at[b, pl.ds(0, 2 * CH)], csem[b])
        pltpu.async_copy(ei_hbm.at[c, pl.ds(base, CH)], dat_v.at[b, pl.ds(2 * CH, CH)], csem[b])

    def _wait_gathers(i, b):
        pltpu.make_async_copy(ei_hbm.at[c, pl.ds(0, 3 * CH)], dat_v.at[b],
                              csem[b]).wait()

    def _snap(b):
        for j in range(CH // L):
            sl = pl.ds(j * L, L)
            sdst_v[b, sl] = dst_v[b, sl]

    def _e2dst(i):
        base = _gbase(i)
        return e2_hbm.at[pl.ds(pl.multiple_of(base // 2, 8), CH // 2),
                         pl.ds(c * D, D)]

    def _wait_outs(i, b):
        @pl.when(i >= 2)
        def _():
            pltpu.make_async_copy(stage_v.at[b], acc_sh.at[sdst_v.at[b]],
                                  gsem[b]).wait()
            pltpu.make_async_copy(e2p_v.at[b], _e2dst(i), osem[b]).wait()

    NJ = DH // L

    def _compute(i, b):
        init = tuple(jnp.zeros((L,), jnp.float32) for _ in range(2 * NJ))

        def _pair_body(q, cr):
            acc = list(cr)
            for r_par in range(2):
                r = q * 2 + r_par
                e2col = r_par * DH
                for j in range(NJ):
                    jL = j * L
                    sl = pl.ds(jL, L)
                    t = dat_v[b, r, sl] + dat_v[b, CH + r, sl] \
                        + dat_v[b, 2 * CH + r, sl]
                    sg = 1.0 / (1.0 + jnp.exp(-t))
                    e2 = t + dat_v[b, 2 * CH + r, pl.ds(DH + jL, L)]
                    e2p_v[b, q, pl.ds(e2col + jL, L)] = e2
                    stage_v[b, r, sl] = dat_v[b, r, pl.ds(DH + jL, L)] * sg
                    stage_v[b, r, pl.ds(DH + jL, L)] = sg
                    acc[j] = acc[j] + e2
                    acc[NJ + j] = acc[NJ + j] + e2 * e2
            return tuple(acc)

        fin = plsc.parallel_loop(0, CH // 2, unroll=2, carry=init)(_pair_body)
        for j in range(NJ):
            plsc.addupdate(st_v.at[0, pl.ds(c0 + j * L, L)], fin[j])
            plsc.addupdate(st_v.at[1, pl.ds(c0 + j * L, L)], fin[NJ + j])

        pltpu.async_copy(e2p_v.at[b], _e2dst(i), osem[b])
        pltpu.async_copy(stage_v.at[b], acc_sh.at[sdst_v.at[b]], gsem[b],
                         add=True)

    _issue_idx(0, 0)
    _wait_idx(0, 0)
    _issue_gathers(0, 0)
    _issue_idx(1, 1)

    def _two(i2, _):
        for b in (1, 0):
            i = 2 * i2 + (1 if b == 1 else 2)
            _wait_idx(i, b)
            _issue_gathers(i, b)
            _wait_gathers(i - 1, 1 - b)
            _wait_outs(i - 1, 1 - b)
            _snap(1 - b)

            @pl.when(i + 1 < KK)
            def _():
                _issue_idx(i + 1, 1 - b)
            _compute(i - 1, 1 - b)
        return 0

    lax.fori_loop(0, (KK - 1) // 2, _two, 0)

    lastb = (KK - 1) % 2
    _wait_gathers(KK - 1, lastb)
    _wait_outs(KK - 1, lastb)
    _snap(lastb)
    _compute(KK - 1, lastb)

    for b in (0, 1):
        pltpu.make_async_copy(stage_v.at[b], acc_sh.at[sdst_v.at[b]],
                              gsem[b]).wait()
        pltpu.make_async_copy(e2p_v.at[b], _e2dst(KK - 1), osem[b]).wait()

    pltpu.sync_copy(st_v, st_hbm.at[c * NS + s])

    plsc.subcore_barrier()
    pltpu.sync_copy(acc_sh.at[pl.ds(s * PERT, PERT)],
                    acc_hbm.at[c, pl.ds(s * PERT, PERT)])


@functools.lru_cache(maxsize=None)
def _sc_kernels():
    mesh = plsc.VectorSubcoreMesh(core_axis_name="c", subcore_axis_name="s",
                                  num_cores=NC, num_subcores=NS)
    edge_pass = pl.kernel(
        _sc_edge_body,
        out_type=(
            jax.ShapeDtypeStruct((E_EDGES // 2, 2 * D), jnp.float32),
            jax.ShapeDtypeStruct((NC, NP, D), jnp.float32),
            jax.ShapeDtypeStruct((NC * NS, 8, D), jnp.float32),
        ),
        mesh=mesh,
        scratch_types=[
            pltpu.VMEM((2, CH), jnp.int32),
            pltpu.VMEM((2, CH), jnp.int32),
            pltpu.VMEM((2, 2 * CH), jnp.int32),
            pltpu.VMEM((2, CH), jnp.int32),
            pltpu.VMEM((2, 3 * CH, D), jnp.float32),
            pltpu.VMEM((2, CH, D), jnp.float32),
            pltpu.VMEM((2, CH // 2, D), jnp.float32),
            pltpu.VMEM((8, D), jnp.float32),
            pltpu.VMEM_SHARED((NP, D), jnp.float32),
        ] + [pltpu.SemaphoreType.DMA] * 10,
    )
    return edge_pass


def kernel(h, e, edge_index, W_A, b_A, W_B, b_B, W_C, b_C, W_D, b_D, W_E, b_E,
           ffh_W1, ffh_b1, ffh_W2, ffh_b2, ffe_W1, ffe_b1, ffe_W2, ffe_b2,
           g1h_g, g1h_b, g1e_g, g1e_b, g2h_g, g2h_b, g2e_g, g2e_b):
    src = edge_index[0]
    dst = edge_index[1]
    row = lambda v: v.reshape(1, D)

    ah, bh, ch, dh = pl.pallas_call(
        _h_pre_body,
        out_shape=[jax.ShapeDtypeStruct((N_NODES, D), jnp.float32)] * 4,
    )(h, row(g1h_g), row(g1h_b), W_A, row(b_A), W_B, row(b_B),
      W_C, row(b_C), W_D, row(b_D))

    BLK = 2000
    grid = E_EDGES // BLK
    estats = pl.pallas_call(
        _colstats_body,
        grid=(grid,),
        in_specs=[pl.BlockSpec((BLK, D), lambda i: (i, 0))],
        out_specs=pl.BlockSpec((8, D), lambda i: (0, 0)),
        out_shape=jax.ShapeDtypeStruct((8, D), jnp.float32),
    )(e)

    ei = pl.pallas_call(
        _e_pre_body,
        grid=(grid,),
        in_specs=[
            pl.BlockSpec((BLK, D), lambda i: (i, 0)),
            pl.BlockSpec((8, D), lambda i: (0, 0)),
            pl.BlockSpec((1, D), lambda i: (0, 0)),
            pl.BlockSpec((1, D), lambda i: (0, 0)),
            pl.BlockSpec((D, D), lambda i: (0, 0)),
            pl.BlockSpec((1, D), lambda i: (0, 0)),
        ],
        out_specs=pl.BlockSpec((NC, BLK, D), lambda i: (0, i, 0)),
        out_shape=jax.ShapeDtypeStruct((NC, E_EDGES, D), jnp.float32),
    )(e, estats, row(g1e_g), row(g1e_b), W_E, row(b_E))

    zpad = jnp.zeros((N_NODES, DH), jnp.float32)
    tbl = jnp.concatenate([
        jnp.concatenate([ch[:, :DH], bh[:, :DH]], axis=1),
        jnp.concatenate([ch[:, DH:], bh[:, DH:]], axis=1),
        jnp.concatenate([dh[:, :DH], zpad], axis=1),
        jnp.concatenate([dh[:, DH:], zpad], axis=1),
    ], axis=0)

    sc_edge = _sc_kernels()
    e2p, acc, st = sc_edge(src, dst, tbl, ei)

    acc_h = jnp.concatenate([acc[0, :N_NODES, :DH], acc[1, :N_NODES, :DH]], axis=1)
    acc_s = jnp.concatenate([acc[0, :N_NODES, DH:], acc[1, :N_NODES, DH:]], axis=1)

    h_out = pl.pallas_call(
        _h_post_body,
        out_shape=jax.ShapeDtypeStruct((N_NODES, D), jnp.float32),
    )(ah, acc_h, acc_s, h, row(g2h_g), row(g2h_b),
      ffh_W1, row(ffh_b1), ffh_W2, row(ffh_b2))

    e_out = pl.pallas_call(
        _e_post_body,
        grid=(grid,),
        in_specs=[
            pl.BlockSpec((BLK // 2, 2 * D), lambda i: (i, 0)),
            pl.BlockSpec((NC * NS, D), lambda i: (0, 0)),
            pl.BlockSpec((NC * NS, D), lambda i: (0, 0)),
            pl.BlockSpec((1, D), lambda i: (0, 0)),
            pl.BlockSpec((1, D), lambda i: (0, 0)),
            pl.BlockSpec((D, D), lambda i: (0, 0)),
            pl.BlockSpec((1, D), lambda i: (0, 0)),
            pl.BlockSpec((D, D), lambda i: (0, 0)),
            pl.BlockSpec((1, D), lambda i: (0, 0)),
        ],
        out_specs=pl.BlockSpec((BLK, D), lambda i: (i, 0)),
        out_shape=jax.ShapeDtypeStruct((E_EDGES, D), jnp.float32),
    )(e2p, st[:, 0, :], st[:, 1, :], row(g2e_g), row(g2e_b),
      ffe_W1, row(ffe_b1), ffe_W2, row(ffe_b2))

    return (h_out, e_out)
